# Initial kernel scaffold; baseline (speedup 1.0000x reference)
import jax
import jax.numpy as jnp
from jax.experimental import pallas as pl


def kernel(i, m):
    raise NotImplementedError("write your pallas kernel here")


# speedup vs baseline: 8.5384x; 8.5384x over previous
import functools

import jax
import jax.numpy as jnp
from jax import lax
from jax.experimental import pallas as pl
from jax.experimental.pallas import tpu as pltpu
from jax.experimental.pallas import tpu_sc as plsc

VOCAB = 100000
D1 = 20
D2 = 32
B = 1024
F = 26

D = D1 * D2
NB = B * F
NW = 32
PER_W = NB // NW
CH = 64
NCH = PER_W // CH


def _make_gather():
    mesh = plsc.VectorSubcoreMesh(core_axis_name="c", subcore_axis_name="s")

    @functools.partial(
        pl.kernel,
        mesh=mesh,
        out_type=jax.ShapeDtypeStruct((NB, D), jnp.float32),
        scratch_types=[
            pltpu.VMEM((NCH, CH), jnp.int32),
            pltpu.VMEM((CH, D), jnp.float32),
            pltpu.VMEM((CH, D), jnp.float32),
            pltpu.SemaphoreType.DMA,
        ],
    )
    def gather_kernel(m_hbm, idx_hbm, out_hbm, idx_v, rows0, rows1, gsem):
        wid = lax.axis_index("s") * 2 + lax.axis_index("c")
        base = wid * PER_W

        pltpu.sync_copy(idx_hbm.at[wid], idx_v)

        bufs = (rows0, rows1)
        pending = pltpu.async_copy(m_hbm.at[idx_v.at[0]], bufs[0], gsem)
        for c in range(NCH):
            cur = bufs[c % 2]
            pending.wait()
            if c + 1 < NCH:
                pending = pltpu.async_copy(
                    m_hbm.at[idx_v.at[c + 1]], bufs[(c + 1) % 2], gsem
                )
            pltpu.sync_copy(cur, out_hbm.at[pl.ds(base + c * CH, CH)])

    return gather_kernel


_gather = _make_gather()


def kernel(i, m):
    idx = i.astype(jnp.int32).reshape(NW, NCH, CH)
    m2 = m.reshape(VOCAB, D)
    out = _gather(m2, idx)
    return out.reshape(B, F, D1, D2)


# speedup vs baseline: 10.7113x; 1.2545x over previous
import functools

import jax
import jax.numpy as jnp
from jax import lax
from jax.experimental import pallas as pl
from jax.experimental.pallas import tpu as pltpu
from jax.experimental.pallas import tpu_sc as plsc

VOCAB = 100000
D1 = 20
D2 = 32
B = 1024
F = 26

D = D1 * D2
NW = 32
DD_PER_W = D // NW
SEG0 = 50048
SEG1 = 49920
TAILW = 128
NBLK = B // 16


def _make_gather():
    mesh = plsc.VectorSubcoreMesh(core_axis_name="c", subcore_axis_name="s")

    @functools.partial(
        pl.kernel,
        mesh=mesh,
        out_type=jax.ShapeDtypeStruct((F, D, B), jnp.float32),
        scratch_types=[
            pltpu.VMEM((SEG0,), jnp.float32),
            pltpu.VMEM((24, TAILW), jnp.float32),
            pltpu.VMEM((F, B), jnp.int32),
            pltpu.VMEM((F, B), jnp.float32),
        ],
        compiler_params=pltpu.CompilerParams(needs_layout_passes=False),
    )
    def gather_kernel(mt_hbm, tail_hbm, idx_hbm, out_hbm, row_v, tail_v, idx_v, obuf):
        wid = lax.axis_index("s") * 2 + lax.axis_index("c")
        dd0 = wid * DD_PER_W
        dd0_al = pl.multiple_of(dd0 - lax.rem(dd0, 8), 8)

        pltpu.sync_copy(idx_hbm, idx_v)
        pltpu.sync_copy(tail_hbm.at[pl.ds(dd0_al, 24)], tail_v)

        def per_dd(k, _):
            dd = dd0 + k
            loc = dd - dd0_al

            pltpu.sync_copy(mt_hbm.at[dd, pl.ds(0, SEG0)], row_v)

            def pass0_f(f, _):
                def pass0_j(j, _):
                    vidx = idx_v[f, pl.ds(j * 16, 16)]
                    msk = vidx < SEG0
                    vals = plsc.load_gather(row_v, [vidx], mask=msk)
                    obuf[f, pl.ds(j * 16, 16)] = vals
                    return 0

                return lax.fori_loop(0, NBLK, pass0_j, 0)

            lax.fori_loop(0, F, pass0_f, 0)

            pltpu.sync_copy(
                mt_hbm.at[dd, pl.ds(SEG0, SEG1)], row_v.at[pl.ds(0, SEG1)]
            )
            row_v[pl.ds(SEG1, 16)] = tail_v[loc, pl.ds(96, 16)]
            row_v[pl.ds(SEG1 + 16, 16)] = tail_v[loc, pl.ds(112, 16)]

            def pass1_f(f, _):
                def pass1_j(j, _):
                    vidx = idx_v[f, pl.ds(j * 16, 16)]
                    msk = vidx >= SEG0
                    vals = plsc.load_gather(row_v, [vidx - SEG0], mask=msk)
                    prev = obuf[f, pl.ds(j * 16, 16)]
                    obuf[f, pl.ds(j * 16, 16)] = jnp.where(msk, vals, prev)
                    return 0

                return lax.fori_loop(0, NBLK, pass1_j, 0)

            lax.fori_loop(0, F, pass1_f, 0)

            pltpu.sync_copy(obuf, out_hbm.at[:, dd])
            return 0

        lax.fori_loop(0, DD_PER_W, per_dd, 0)

    return gather_kernel


_gather = _make_gather()


def kernel(i, m):
    mt = jnp.transpose(m, (1, 2, 0)).reshape(D, VOCAB)
    mtail = jnp.transpose(m[VOCAB - TAILW :], (1, 2, 0)).reshape(D, TAILW)
    iT = jnp.transpose(i.astype(jnp.int32), (1, 0))
    out_t = _gather(mt, mtail, iT)
    return jnp.transpose(out_t.reshape(F, D1, D2, B), (3, 0, 1, 2))
